# baseline (speedup 1.0000x reference)
import jax
import jax.numpy as jnp
from jax.experimental import pallas as pl
from jax.experimental.pallas import tpu as pltpu

_BLOCK_T = 512
_BLOCK_K = 1024


def _router_kernel(x_ref, wt_ref, b_ref, mask_ref, probs_ref, acc_ref):
    k = pl.program_id(1)
    nk = pl.num_programs(1)
    part = jnp.dot(x_ref[...], wt_ref[...], preferred_element_type=jnp.float32)

    @pl.when(k == 0)
    def _init():
        acc_ref[...] = part

    @pl.when(k > 0)
    def _accum():
        acc_ref[...] = acc_ref[...] + part

    @pl.when(k == nk - 1)
    def _finish():
        logits = acc_ref[...] + b_ref[...]
        m = jnp.max(logits, axis=-1, keepdims=True)
        e = jnp.exp(logits - m)
        probs_ref[...] = e / jnp.sum(e, axis=-1, keepdims=True)
        E = logits.shape[-1]
        iota = jax.lax.broadcasted_iota(jnp.int32, logits.shape, 1)
        first = jnp.min(jnp.where(logits == m, iota, E), axis=-1, keepdims=True)
        mask_ref[...] = (iota == first).astype(jnp.float32)


def kernel(x, W, b):
    T, D = x.shape
    E = W.shape[0]
    wt = W.T
    b2 = b.reshape(1, E)
    grid = (T // _BLOCK_T, D // _BLOCK_K)
    mask, probs = pl.pallas_call(
        _router_kernel,
        grid=grid,
        in_specs=[
            pl.BlockSpec((_BLOCK_T, _BLOCK_K), lambda i, k: (i, k)),
            pl.BlockSpec((_BLOCK_K, E), lambda i, k: (k, 0)),
            pl.BlockSpec((1, E), lambda i, k: (0, 0)),
        ],
        out_specs=[
            pl.BlockSpec((_BLOCK_T, E), lambda i, k: (i, 0)),
            pl.BlockSpec((_BLOCK_T, E), lambda i, k: (i, 0)),
        ],
        out_shape=[
            jax.ShapeDtypeStruct((T, E), jnp.float32),
            jax.ShapeDtypeStruct((T, E), jnp.float32),
        ],
        scratch_shapes=[
            pltpu.VMEM((_BLOCK_T, E), jnp.float32),
        ],
        compiler_params=pltpu.CompilerParams(
            dimension_semantics=("parallel", "arbitrary"),
        ),
    )(x, wt, b2)
    return (mask, probs)


# speedup vs baseline: 1.6690x; 1.6690x over previous
import functools

import jax
import jax.numpy as jnp
from jax import lax
from jax.experimental import pallas as pl
from jax.experimental.pallas import tpu as pltpu
from jax.experimental.pallas import tpu_sc as plsc

_BLOCK_T = 1024

_NC = 2
_NS = 16
_NW = _NC * _NS
_L = 16
_RCHUNK = 256


def _router_tc_kernel(x_ref, wt_ref, b_ref, probs_ref, idx_ref):
    logits = jnp.dot(x_ref[...], wt_ref[...],
                     preferred_element_type=jnp.float32)
    logits = logits + b_ref[...]
    m = jnp.max(logits, axis=-1, keepdims=True)
    e = jnp.exp(logits - m)
    probs_ref[...] = e / jnp.sum(e, axis=-1, keepdims=True)
    E = logits.shape[-1]
    iota = jax.lax.broadcasted_iota(jnp.int32, logits.shape, 1)
    idx_ref[...] = jnp.min(jnp.where(logits == m, iota, E), axis=-1,
                           keepdims=True)


def _mask_sc_kernel(E, rows_w, idx_hbm, mask_hbm, idx_v, mk_v):
    wid = lax.axis_index("s") * _NC + lax.axis_index("c")
    base = wid * rows_w
    nchunks = rows_w // _RCHUNK

    def chunk_body(c, _):
        row0 = base + c * _RCHUNK
        pltpu.sync_copy(idx_hbm.at[pl.ds(row0, _RCHUNK)], idx_v)

        def zero_body(r, _):
            for cc in range(E // _L):
                mk_v[r, pl.ds(cc * _L, _L)] = jnp.zeros((_L,), jnp.float32)
            return 0

        lax.fori_loop(0, _RCHUNK, zero_body, 0)

        ones = jnp.ones((_L,), jnp.float32)
        lane = lax.iota(jnp.int32, _L)

        def scat_body(g, _):
            rows = g * _L + lane
            cols = idx_v[pl.ds(g * _L, _L)]
            plsc.store_scatter(mk_v, [rows, cols], ones)
            return 0

        lax.fori_loop(0, _RCHUNK // _L, scat_body, 0)
        pltpu.sync_copy(mk_v, mask_hbm.at[pl.ds(row0, _RCHUNK), :])
        return 0

    lax.fori_loop(0, nchunks, chunk_body, 0)


def kernel(x, W, b):
    T, D = x.shape
    E = W.shape[0]
    wt = W.T
    b2 = b.reshape(1, E)
    grid = (T // _BLOCK_T,)
    probs, idx = pl.pallas_call(
        _router_tc_kernel,
        grid=grid,
        in_specs=[
            pl.BlockSpec((_BLOCK_T, D), lambda i: (i, 0)),
            pl.BlockSpec((D, E), lambda i: (0, 0)),
            pl.BlockSpec((1, E), lambda i: (0, 0)),
        ],
        out_specs=[
            pl.BlockSpec((_BLOCK_T, E), lambda i: (i, 0)),
            pl.BlockSpec((_BLOCK_T, 1), lambda i: (i, 0)),
        ],
        out_shape=[
            jax.ShapeDtypeStruct((T, E), jnp.float32),
            jax.ShapeDtypeStruct((T, 1), jnp.int32),
        ],
    )(x, wt, b2)

    rows_w = T // _NW
    sc_mask = pl.kernel(
        functools.partial(_mask_sc_kernel, E, rows_w),
        out_type=jax.ShapeDtypeStruct((T, E), jnp.float32),
        mesh=plsc.VectorSubcoreMesh(core_axis_name="c", subcore_axis_name="s"),
        scratch_types=[
            pltpu.VMEM((_RCHUNK,), jnp.int32),
            pltpu.VMEM((_RCHUNK, E), jnp.float32),
        ],
        compiler_params=pltpu.CompilerParams(needs_layout_passes=False),
    )
    mask = sc_mask(idx.reshape(T))
    return (mask, probs)


# speedup vs baseline: 1.8931x; 1.1343x over previous
import jax
import jax.numpy as jnp
from jax.experimental import pallas as pl

_BLOCK_T = 1024


def _router_kernel(x_ref, wt_ref, b_ref, mask_ref, probs_ref):
    logits = jnp.dot(x_ref[...], wt_ref[...],
                     preferred_element_type=jnp.float32)
    logits = logits + b_ref[...]
    m = jnp.max(logits, axis=-1, keepdims=True)
    e = jnp.exp(logits - m)
    probs_ref[...] = e / jnp.sum(e, axis=-1, keepdims=True)
    E = logits.shape[-1]
    iota = jax.lax.broadcasted_iota(jnp.int32, logits.shape, 1)
    first = jnp.min(jnp.where(logits == m, iota, E), axis=-1, keepdims=True)
    mask_ref[...] = (iota == first).astype(jnp.float32)


def kernel(x, W, b):
    T, D = x.shape
    E = W.shape[0]
    wt = W.T
    b2 = b.reshape(1, E)
    grid = (T // _BLOCK_T,)
    mask, probs = pl.pallas_call(
        _router_kernel,
        grid=grid,
        in_specs=[
            pl.BlockSpec((_BLOCK_T, D), lambda i: (i, 0)),
            pl.BlockSpec((D, E), lambda i: (0, 0)),
            pl.BlockSpec((1, E), lambda i: (0, 0)),
        ],
        out_specs=[
            pl.BlockSpec((_BLOCK_T, E), lambda i: (i, 0)),
            pl.BlockSpec((_BLOCK_T, E), lambda i: (i, 0)),
        ],
        out_shape=[
            jax.ShapeDtypeStruct((T, E), jnp.float32),
            jax.ShapeDtypeStruct((T, E), jnp.float32),
        ],
    )(x, wt, b2)
    return (mask, probs)
